# baseline (speedup 1.0000x reference)
import functools

import jax
import jax.numpy as jnp
from jax import lax
from jax.experimental import pallas as pl
from jax.experimental.pallas import tpu as pltpu
from jax.experimental.pallas import tpu_sc as plsc

NC = 2
NS = 16
NW = NC * NS


def _make_sc_gather(d, base0, nk, chunk):
    b_per_w = nk // NW
    nch = b_per_w // chunk
    mesh = plsc.VectorSubcoreMesh(core_axis_name="c", subcore_axis_name="s")

    @functools.partial(
        pl.kernel,
        mesh=mesh,
        out_type=jax.ShapeDtypeStruct((nk, d), jnp.float32),
        scratch_types=[
            pltpu.VMEM((b_per_w,), jnp.int32),
            pltpu.VMEM((chunk, d), jnp.float32),
            pltpu.VMEM((chunk, d), jnp.float32),
            pltpu.VMEM((chunk, d), jnp.float32),
            pltpu.SemaphoreType.DMA,
            pltpu.SemaphoreType.DMA,
            pltpu.SemaphoreType.DMA,
            pltpu.SemaphoreType.DMA,
            pltpu.SemaphoreType.DMA,
            pltpu.SemaphoreType.DMA,
        ],
    )
    def sc_gather(table_hbm, idx_hbm, out_hbm, idx_v, rows0, rows1, rows2,
                  g0, g1, g2, w0, w1, w2):
        wid = lax.axis_index("s") * NC + lax.axis_index("c")
        base = wid * b_per_w
        pltpu.sync_copy(idx_hbm.at[pl.ds(base0 + base, b_per_w)], idx_v)
        rows = (rows0, rows1, rows2)
        gsem = (g0, g1, g2)
        wsem = (w0, w1, w2)
        gh = [None, None, None]
        wh = [None, None, None]

        def write(c):
            pj = c % 3
            gh[pj].wait()
            wh[pj] = pltpu.async_copy(
                rows[pj],
                out_hbm.at[pl.ds(base + c * chunk, chunk)],
                wsem[pj])

        for c in range(nch):
            bi = c % 3
            if wh[bi] is not None:
                wh[bi].wait()
            gh[bi] = pltpu.async_copy(
                table_hbm.at[idx_v.at[pl.ds(c * chunk, chunk)]],
                rows[bi], gsem[bi])
            if c >= 2:
                write(c - 2)
        write(nch - 2)
        write(nch - 1)
        for h in wh:
            if h is not None:
                h.wait()

    return sc_gather


def _ln_math(g_ref, pos_ref, seg_ref, w_ref, o_ref):
    h = g_ref[...] + pos_ref[...]
    t0 = w_ref[0:1, :]
    t1 = w_ref[1:2, :]
    h = h + t0 + seg_ref[...].astype(jnp.float32) * (t1 - t0)
    mean = jnp.mean(h, axis=1, keepdims=True)
    c = h - mean
    var = jnp.mean(c * c, axis=1, keepdims=True)
    o_ref[...] = (c * lax.rsqrt(var + 1e-12) * w_ref[2:3, :]
                  + w_ref[3:4, :])


def _ln_body_first(g_ref, pos_ref, seg_ref, w_ref, o_ref):
    _ln_math(g_ref, pos_ref, seg_ref, w_ref, o_ref)


def _ln_body_alias(buf_ref, g_ref, pos_ref, seg_ref, w_ref, o_ref):
    del buf_ref
    _ln_math(g_ref, pos_ref, seg_ref, w_ref, o_ref)


def kernel(input_ids, segment_ids, W_word, W_pos, W_type, gamma, beta):
    b, s = input_ids.shape
    vocab, d = W_word.shape
    n = b * s

    sizes = (1024, 1024, 1536, 512)
    offs = (0, 1024, 2048, 3584)
    tss = (1024, 1024, 512, 512)

    ids = input_ids.astype(jnp.int32)
    ids_perm = jnp.concatenate(
        [ids[:, off:off + sk].reshape(b * sk)
         for off, sk in zip(offs, sizes)])
    seg_f = segment_ids.reshape(n, 1).astype(jnp.int8)
    w_small = jnp.concatenate(
        [W_type, gamma.reshape(1, d), beta.reshape(1, d)], axis=0)

    staged = [_make_sc_gather(d, b * off, b * sk, chunk=32)(W_word, ids_perm)
              for off, sk in zip(offs, sizes)]

    out_buf = None
    for kk, (off, sk, ts) in enumerate(zip(offs, sizes, tss)):
        nblk = s // ts
        pb = sk // ts
        ob = off // ts
        g_spec = pl.BlockSpec((ts, d), lambda j, i, pb=pb: (i * pb + j, 0))
        pos_spec = pl.BlockSpec((ts, d), lambda j, i, ob=ob: (ob + j, 0))
        seg_spec = pl.BlockSpec(
            (ts, 1), lambda j, i, ob=ob: (i * nblk + ob + j, 0))
        w_spec = pl.BlockSpec((4, d), lambda j, i: (0, 0))
        out_spec = pl.BlockSpec(
            (ts, d), lambda j, i, ob=ob: (i * nblk + ob + j, 0))
        if out_buf is None:
            out_buf = pl.pallas_call(
                _ln_body_first,
                grid=(pb, b),
                in_specs=[g_spec, pos_spec, seg_spec, w_spec],
                out_specs=out_spec,
                out_shape=jax.ShapeDtypeStruct((n, d), jnp.float32),
            )(staged[kk], W_pos, seg_f, w_small)
        else:
            out_buf = pl.pallas_call(
                _ln_body_alias,
                grid=(pb, b),
                in_specs=[pl.BlockSpec((8, 128), lambda j, i: (0, 0)),
                          g_spec, pos_spec, seg_spec, w_spec],
                out_specs=out_spec,
                out_shape=jax.ShapeDtypeStruct((n, d), jnp.float32),
                input_output_aliases={0: 0},
            )(out_buf, staged[kk], W_pos, seg_f, w_small)

    return out_buf.reshape(b, s, d)


# speedup vs baseline: 1.0340x; 1.0340x over previous
import functools

import jax
import jax.numpy as jnp
from jax import lax
from jax.experimental import pallas as pl
from jax.experimental.pallas import tpu as pltpu
from jax.experimental.pallas import tpu_sc as plsc

NC = 2
NS = 16
NW = NC * NS


def _make_sc_gather(d, base0, nk, chunk):
    b_per_w = nk // NW
    nch = b_per_w // chunk
    mesh = plsc.VectorSubcoreMesh(core_axis_name="c", subcore_axis_name="s")

    @functools.partial(
        pl.kernel,
        mesh=mesh,
        out_type=jax.ShapeDtypeStruct((nk, d), jnp.float32),
        scratch_types=[
            pltpu.VMEM((b_per_w,), jnp.int32),
            pltpu.VMEM((chunk, d), jnp.float32),
            pltpu.VMEM((chunk, d), jnp.float32),
            pltpu.VMEM((chunk, d), jnp.float32),
            pltpu.SemaphoreType.DMA,
            pltpu.SemaphoreType.DMA,
            pltpu.SemaphoreType.DMA,
            pltpu.SemaphoreType.DMA,
            pltpu.SemaphoreType.DMA,
            pltpu.SemaphoreType.DMA,
        ],
    )
    def sc_gather(table_hbm, idx_hbm, out_hbm, idx_v, rows0, rows1, rows2,
                  g0, g1, g2, w0, w1, w2):
        wid = lax.axis_index("s") * NC + lax.axis_index("c")
        base = wid * b_per_w
        pltpu.sync_copy(idx_hbm.at[pl.ds(base0 + base, b_per_w)], idx_v)
        rows = (rows0, rows1, rows2)
        gsem = (g0, g1, g2)
        wsem = (w0, w1, w2)
        gh = [None, None, None]
        wh = [None, None, None]

        def write(c):
            pj = c % 3
            gh[pj].wait()
            wh[pj] = pltpu.async_copy(
                rows[pj],
                out_hbm.at[pl.ds(base + c * chunk, chunk)],
                wsem[pj])

        for c in range(nch):
            bi = c % 3
            if wh[bi] is not None:
                wh[bi].wait()
            gh[bi] = pltpu.async_copy(
                table_hbm.at[idx_v.at[pl.ds(c * chunk, chunk)]],
                rows[bi], gsem[bi])
            if c >= 2:
                write(c - 2)
        write(nch - 2)
        write(nch - 1)
        for h in wh:
            if h is not None:
                h.wait()

    return sc_gather


def _ln_math(g_ref, pos_ref, seg_ref, w_ref, o_ref):
    h = g_ref[...] + pos_ref[...]
    t0 = w_ref[0:1, :]
    t1 = w_ref[1:2, :]
    h = h + t0 + seg_ref[...].astype(jnp.float32) * (t1 - t0)
    mean = jnp.mean(h, axis=1, keepdims=True)
    c = h - mean
    var = jnp.mean(c * c, axis=1, keepdims=True)
    o_ref[...] = (c * lax.rsqrt(var + 1e-12) * w_ref[2:3, :]
                  + w_ref[3:4, :])


def _ln_body_first(g_ref, pos_ref, seg_ref, w_ref, o_ref):
    _ln_math(g_ref, pos_ref, seg_ref, w_ref, o_ref)


def _ln_body_alias(buf_ref, g_ref, pos_ref, seg_ref, w_ref, o_ref):
    del buf_ref
    _ln_math(g_ref, pos_ref, seg_ref, w_ref, o_ref)


def kernel(input_ids, segment_ids, W_word, W_pos, W_type, gamma, beta):
    b, s = input_ids.shape
    vocab, d = W_word.shape
    n = b * s

    sizes = (1024, 1024, 1024, 1024)
    offs = (0, 1024, 2048, 3072)
    tss = (1024, 1024, 1024, 1024)

    ids = input_ids.astype(jnp.int32)
    ids_perm = jnp.concatenate(
        [ids[:, off:off + sk].reshape(b * sk)
         for off, sk in zip(offs, sizes)])
    seg_f = segment_ids.reshape(n, 1).astype(jnp.int8)
    w_small = jnp.concatenate(
        [W_type, gamma.reshape(1, d), beta.reshape(1, d)], axis=0)

    staged = [_make_sc_gather(d, b * off, b * sk, chunk=32)(W_word, ids_perm)
              for off, sk in zip(offs, sizes)]

    out_buf = None
    for kk, (off, sk, ts) in enumerate(zip(offs, sizes, tss)):
        nblk = s // ts
        pb = sk // ts
        ob = off // ts
        g_spec = pl.BlockSpec((ts, d), lambda j, i, pb=pb: (i * pb + j, 0))
        pos_spec = pl.BlockSpec((ts, d), lambda j, i, ob=ob: (ob + j, 0))
        seg_spec = pl.BlockSpec(
            (ts, 1), lambda j, i, ob=ob: (i * nblk + ob + j, 0))
        w_spec = pl.BlockSpec((4, d), lambda j, i: (0, 0))
        out_spec = pl.BlockSpec(
            (ts, d), lambda j, i, ob=ob: (i * nblk + ob + j, 0))
        if out_buf is None:
            out_buf = pl.pallas_call(
                _ln_body_first,
                grid=(pb, b),
                in_specs=[g_spec, pos_spec, seg_spec, w_spec],
                out_specs=out_spec,
                out_shape=jax.ShapeDtypeStruct((n, d), jnp.float32),
            )(staged[kk], W_pos, seg_f, w_small)
        else:
            out_buf = pl.pallas_call(
                _ln_body_alias,
                grid=(pb, b),
                in_specs=[pl.BlockSpec((8, 128), lambda j, i: (0, 0)),
                          g_spec, pos_spec, seg_spec, w_spec],
                out_specs=out_spec,
                out_shape=jax.ShapeDtypeStruct((n, d), jnp.float32),
                input_output_aliases={0: 0},
            )(out_buf, staged[kk], W_pos, seg_f, w_small)

    return out_buf.reshape(b, s, d)


# speedup vs baseline: 1.0395x; 1.0053x over previous
import functools

import jax
import jax.numpy as jnp
from jax import lax
from jax.experimental import pallas as pl
from jax.experimental.pallas import tpu as pltpu
from jax.experimental.pallas import tpu_sc as plsc

NC = 2
NS = 16
NW = NC * NS


def _make_sc_gather(d, base0, nk, chunk, nbuf=3, lag=2):
    b_per_w = nk // NW
    nch = b_per_w // chunk
    mesh = plsc.VectorSubcoreMesh(core_axis_name="c", subcore_axis_name="s")

    @functools.partial(
        pl.kernel,
        mesh=mesh,
        out_type=jax.ShapeDtypeStruct((nk, d), jnp.float32),
        scratch_types=(
            [pltpu.VMEM((b_per_w,), jnp.int32)]
            + [pltpu.VMEM((chunk, d), jnp.float32)] * nbuf
            + [pltpu.SemaphoreType.DMA] * (2 * nbuf)
        ),
    )
    def sc_gather(table_hbm, idx_hbm, out_hbm, idx_v, *scr):
        rows = scr[:nbuf]
        gsem = scr[nbuf:2 * nbuf]
        wsem = scr[2 * nbuf:]
        wid = lax.axis_index("s") * NC + lax.axis_index("c")
        base = wid * b_per_w
        pltpu.sync_copy(idx_hbm.at[pl.ds(base0 + base, b_per_w)], idx_v)
        gh = [None] * nbuf
        wh = [None] * nbuf

        def write(c):
            pj = c % nbuf
            gh[pj].wait()
            wh[pj] = pltpu.async_copy(
                rows[pj],
                out_hbm.at[pl.ds(base + c * chunk, chunk)],
                wsem[pj])

        for c in range(nch):
            bi = c % nbuf
            if wh[bi] is not None:
                wh[bi].wait()
            gh[bi] = pltpu.async_copy(
                table_hbm.at[idx_v.at[pl.ds(c * chunk, chunk)]],
                rows[bi], gsem[bi])
            if c >= lag:
                write(c - lag)
        for c in range(max(0, nch - lag), nch):
            write(c)
        for h in wh:
            if h is not None:
                h.wait()

    return sc_gather


def _ln_math(g_ref, pos_ref, seg_ref, w_ref, o_ref):
    h = g_ref[...] + pos_ref[...]
    t0 = w_ref[0:1, :]
    t1 = w_ref[1:2, :]
    h = h + t0 + seg_ref[...].astype(jnp.float32) * (t1 - t0)
    mean = jnp.mean(h, axis=1, keepdims=True)
    c = h - mean
    var = jnp.mean(c * c, axis=1, keepdims=True)
    o_ref[...] = (c * lax.rsqrt(var + 1e-12) * w_ref[2:3, :]
                  + w_ref[3:4, :])


def _ln_body_first(g_ref, pos_ref, seg_ref, w_ref, o_ref):
    _ln_math(g_ref, pos_ref, seg_ref, w_ref, o_ref)


def _ln_body_alias(buf_ref, g_ref, pos_ref, seg_ref, w_ref, o_ref):
    del buf_ref
    _ln_math(g_ref, pos_ref, seg_ref, w_ref, o_ref)


def kernel(input_ids, segment_ids, W_word, W_pos, W_type, gamma, beta):
    b, s = input_ids.shape
    vocab, d = W_word.shape
    n = b * s

    sizes = (1024, 1024, 1024, 1024)
    offs = (0, 1024, 2048, 3072)
    tss = (1024, 1024, 1024, 1024)

    ids = input_ids.astype(jnp.int32)
    ids_perm = jnp.concatenate(
        [ids[:, off:off + sk].reshape(b * sk)
         for off, sk in zip(offs, sizes)])
    seg_f = segment_ids.reshape(n, 1).astype(jnp.int8)
    w_small = jnp.concatenate(
        [W_type, gamma.reshape(1, d), beta.reshape(1, d)], axis=0)

    staged = [_make_sc_gather(d, b * off, b * sk, chunk=16, nbuf=6,
                              lag=4)(W_word, ids_perm)
              for off, sk in zip(offs, sizes)]

    out_buf = None
    for kk, (off, sk, ts) in enumerate(zip(offs, sizes, tss)):
        nblk = s // ts
        pb = sk // ts
        ob = off // ts
        g_spec = pl.BlockSpec((ts, d), lambda j, i, pb=pb: (i * pb + j, 0))
        pos_spec = pl.BlockSpec((ts, d), lambda j, i, ob=ob: (ob + j, 0))
        seg_spec = pl.BlockSpec(
            (ts, 1), lambda j, i, ob=ob: (i * nblk + ob + j, 0))
        w_spec = pl.BlockSpec((4, d), lambda j, i: (0, 0))
        out_spec = pl.BlockSpec(
            (ts, d), lambda j, i, ob=ob: (i * nblk + ob + j, 0))
        if out_buf is None:
            out_buf = pl.pallas_call(
                _ln_body_first,
                grid=(pb, b),
                in_specs=[g_spec, pos_spec, seg_spec, w_spec],
                out_specs=out_spec,
                out_shape=jax.ShapeDtypeStruct((n, d), jnp.float32),
            )(staged[kk], W_pos, seg_f, w_small)
        else:
            out_buf = pl.pallas_call(
                _ln_body_alias,
                grid=(pb, b),
                in_specs=[pl.BlockSpec((8, 128), lambda j, i: (0, 0)),
                          g_spec, pos_spec, seg_spec, w_spec],
                out_specs=out_spec,
                out_shape=jax.ShapeDtypeStruct((n, d), jnp.float32),
                input_output_aliases={0: 0},
            )(out_buf, staged[kk], W_pos, seg_f, w_small)

    return out_buf.reshape(b, s, d)
